# baseline (speedup 1.0000x reference)
import functools

import jax
import jax.numpy as jnp
from jax.experimental import pallas as pl
from jax.experimental.pallas import tpu as pltpu

EPS = 1e-5


def _fold(layers):
    out = []
    for (W, b, g, be) in layers:
        s = g / jnp.sqrt(1.0 + EPS)
        out.append(((W * s[:, None]).T, (b * s + be)[None, :]))
    return out


def _fps_body(ptsT_ref, out_ref, *, npoint):
    B = ptsT_ref.shape[0]
    N = ptsT_ref.shape[2]
    x = ptsT_ref[:, 0, :]
    y = ptsT_ref[:, 1, :]
    z = ptsT_ref[:, 2, :]
    P = ptsT_ref[...].reshape(B * 3, N)
    iota3 = jax.lax.broadcasted_iota(jnp.int32, (B * 3, N), 1)

    def body(i, carry):
        dist, far = carry
        far3 = jnp.broadcast_to(far[:, None, :], (B, 3, 1)).reshape(B * 3, 1)
        c = jnp.sum(jnp.where(iota3 == far3, P, 0.0), axis=1,
                    keepdims=True).reshape(B, 3)
        cx, cy, cz = c[:, 0:1], c[:, 1:2], c[:, 2:3]
        d = (x - cx) ** 2 + (y - cy) ** 2 + (z - cz) ** 2
        dist = jnp.minimum(dist, d)
        far_new = jnp.argmax(dist, axis=1).astype(jnp.int32)[:, None]
        out_ref[pl.ds(i, 1), 0, :, :] = c[None]
        return dist, far_new

    dist0 = jnp.full((B, N), 1e10, jnp.float32)
    far0 = jnp.zeros((B, 1), jnp.int32)
    jax.lax.fori_loop(0, npoint, body, (dist0, far0))


def _fps(ptsT, npoint):
    B, _, N = ptsT.shape
    out = pl.pallas_call(
        functools.partial(_fps_body, npoint=npoint),
        out_shape=jax.ShapeDtypeStruct((npoint, 1, B, 3), jnp.float32),
    )(ptsT)
    return jnp.transpose(out.reshape(npoint, B, 3), (1, 0, 2))


def _sa_body(pts_ref, ptsT_ref, feat_ref, nx_ref, w1_ref, b1_ref, w2_ref,
             b2_ref, w3_ref, b3_ref, out_ref, *, r2, K):
    pts = pts_ref[0]
    ptsT = ptsT_ref[0]
    feat = feat_ref[0]
    nx = nx_ref[0]
    N = pts.shape[0]
    S = nx.shape[0]
    w1 = w1_ref[...]
    b1 = b1_ref[...]

    Y = jnp.dot(jnp.concatenate([pts, feat], axis=1), w1,
                preferred_element_type=jnp.float32)
    Z = jnp.dot(nx, w1[0:3, :],
                preferred_element_type=jnp.float32)

    s_new = jnp.sum(nx * nx, axis=1, keepdims=True)
    s_src = jnp.sum(ptsT * ptsT, axis=0, keepdims=True)
    cross = jnp.dot(nx, ptsT, preferred_element_type=jnp.float32)
    sqd = (s_new + s_src) - 2.0 * cross

    mask = sqd <= r2
    r = mask.astype(jnp.int16)
    sh = 1
    while sh < N:
        r = r + jnp.concatenate(
            [jnp.zeros((S, sh), jnp.int16), r[:, : N - sh]], axis=1)
        sh *= 2
    cnt = r[:, N - 1: N].astype(jnp.int32)

    t = jnp.where(mask, r, jnp.int16(0))
    tb = t.astype(jnp.bfloat16)
    Yb = Y.astype(jnp.bfloat16)
    one_b = jnp.bfloat16(1.0)
    zero_b = jnp.bfloat16(0.0)
    GRP = 8
    Zg = jnp.concatenate([Z - b1] * GRP, axis=0)
    hs = []
    for k0 in range(0, K, GRP):
        oh = jnp.concatenate(
            [jnp.where(tb == jnp.bfloat16(k + 1), one_b, zero_b)
             for k in range(k0, k0 + GRP)], axis=0)
        g = jnp.dot(oh, Yb, preferred_element_type=jnp.float32)
        hs.append(jnp.maximum(g - Zg, 0.0))
    H = jnp.concatenate(hs, axis=0)

    H = jnp.maximum(
        jnp.dot(H.astype(jnp.bfloat16), w2_ref[...].astype(jnp.bfloat16),
                preferred_element_type=jnp.float32)
        + b2_ref[...], 0.0)
    H = jnp.maximum(
        jnp.dot(H.astype(jnp.bfloat16), w3_ref[...].astype(jnp.bfloat16),
                preferred_element_type=jnp.float32)
        + b3_ref[...], 0.0)
    C3 = H.shape[1]
    H = H.reshape(K, S, C3)
    kio = jax.lax.broadcasted_iota(jnp.int32, (K, S, 1), 0)
    out_ref[0] = jnp.max(jnp.where(cnt[None, :, :] > kio, H, 0.0), axis=0)


def _sa(pts, feats, new_xyz, layers, radius, K):
    B, N, _ = pts.shape
    Cf = feats.shape[2]
    S = new_xyz.shape[1]
    (w1, b1), (w2, b2), (w3, b3) = layers
    C3 = w3.shape[1]
    ptsT = jnp.transpose(pts, (0, 2, 1))
    return pl.pallas_call(
        functools.partial(_sa_body, r2=radius ** 2, K=K),
        grid=(B,),
        in_specs=[
            pl.BlockSpec((1, N, 3), lambda b: (b, 0, 0)),
            pl.BlockSpec((1, 3, N), lambda b: (b, 0, 0)),
            pl.BlockSpec((1, N, Cf), lambda b: (b, 0, 0)),
            pl.BlockSpec((1, S, 3), lambda b: (b, 0, 0)),
            pl.BlockSpec(w1.shape, lambda b: (0, 0)),
            pl.BlockSpec(b1.shape, lambda b: (0, 0)),
            pl.BlockSpec(w2.shape, lambda b: (0, 0)),
            pl.BlockSpec(b2.shape, lambda b: (0, 0)),
            pl.BlockSpec(w3.shape, lambda b: (0, 0)),
            pl.BlockSpec(b3.shape, lambda b: (0, 0)),
        ],
        out_specs=pl.BlockSpec((1, S, C3), lambda b: (b, 0, 0)),
        out_shape=jax.ShapeDtypeStruct((B, S, C3), jnp.float32),
    )(pts, ptsT, feats, new_xyz, w1, b1, w2, b2, w3, b3)


def _sa3_body(xyz_ref, feat_ref, w1_ref, b1_ref, w2_ref, b2_ref, w3_ref,
              b3_ref, out_ref, *, B, M):
    x = jnp.concatenate([xyz_ref[...], feat_ref[...]], axis=1)
    h = jnp.maximum(
        jnp.dot(x, w1_ref[...], preferred_element_type=jnp.float32)
        + b1_ref[...], 0.0)
    h = jnp.maximum(
        jnp.dot(h, w2_ref[...], preferred_element_type=jnp.float32)
        + b2_ref[...], 0.0)
    h = jnp.maximum(
        jnp.dot(h, w3_ref[...], preferred_element_type=jnp.float32)
        + b3_ref[...], 0.0)
    out_ref[...] = jnp.max(h.reshape(B, M, h.shape[1]), axis=1)


def _sa3(l_xyz, l_points, layers):
    B, M, _ = l_xyz.shape
    (w1, b1), (w2, b2), (w3, b3) = layers
    C3 = w3.shape[1]
    return pl.pallas_call(
        functools.partial(_sa3_body, B=B, M=M),
        out_shape=jax.ShapeDtypeStruct((B, C3), jnp.float32),
    )(l_xyz.reshape(B * M, 3), l_points.reshape(B * M, -1),
      w1, b1, w2, b2, w3, b3)


def kernel(xyz, params):
    B = xyz.shape[0]
    l0_xyz = jnp.transpose(xyz, (0, 2, 1))
    sa1 = _fold(params['sa1'])
    sa2 = _fold(params['sa2'])
    sa3 = _fold(params['sa3'])

    l1_xyz = _fps(xyz, 512)
    l1_points = _sa(l0_xyz, l0_xyz, l1_xyz, sa1, 0.2, 32)
    l2_xyz = _fps(jnp.transpose(l1_xyz, (0, 2, 1)), 128)
    l2_points = _sa(l1_xyz, l1_points, l2_xyz, sa2, 0.4, 64)
    return _sa3(l2_xyz, l2_points, sa3)


# speedup vs baseline: 1.0172x; 1.0172x over previous
import functools

import jax
import jax.numpy as jnp
from jax.experimental import pallas as pl

EPS = 1e-5


def _fold(layers):
    out = []
    for (W, b, g, be) in layers:
        s = g / jnp.sqrt(1.0 + EPS)
        out.append(((W * s[:, None]).T, (b * s + be)[None, :]))
    return out


def _fps_body(ptsT_ref, out_ref, *, npoint):
    B = ptsT_ref.shape[0]
    N = ptsT_ref.shape[2]
    x = ptsT_ref[:, 0, :]
    y = ptsT_ref[:, 1, :]
    z = ptsT_ref[:, 2, :]
    iota = jax.lax.broadcasted_iota(jnp.int32, (B, N), 1)

    def body(i, carry):
        dist, far = carry
        sel = iota == far
        cx = jnp.sum(jnp.where(sel, x, 0.0), axis=1, keepdims=True)
        cy = jnp.sum(jnp.where(sel, y, 0.0), axis=1, keepdims=True)
        cz = jnp.sum(jnp.where(sel, z, 0.0), axis=1, keepdims=True)
        d = (x - cx) ** 2 + (y - cy) ** 2 + (z - cz) ** 2
        dist = jnp.minimum(dist, d)
        far_new = jnp.argmax(dist, axis=1).astype(jnp.int32)[:, None]
        out_ref[pl.ds(i, 1), 0, :, :] = jnp.concatenate([cx, cy, cz],
                                                        axis=1)[None]
        return dist, far_new

    dist0 = jnp.full((B, N), 1e10, jnp.float32)
    far0 = jnp.zeros((B, 1), jnp.int32)
    jax.lax.fori_loop(0, npoint, body, (dist0, far0))


def _fps(ptsT, npoint):
    B, _, N = ptsT.shape
    out = pl.pallas_call(
        functools.partial(_fps_body, npoint=npoint),
        out_shape=jax.ShapeDtypeStruct((npoint, 1, B, 3), jnp.float32),
    )(ptsT)
    return jnp.transpose(out.reshape(npoint, B, 3), (1, 0, 2))


def _sa_body(pts_ref, ptsT_ref, feat_ref, nx_ref, w1_ref, b1_ref, w2_ref,
             b2_ref, w3_ref, b3_ref, out_ref, *, r2, K):
    pts = pts_ref[0]
    ptsT = ptsT_ref[0]
    feat = feat_ref[0]
    nx = nx_ref[0]
    N = pts.shape[0]
    S = nx.shape[0]
    w1 = w1_ref[...]
    b1 = b1_ref[...]

    Y = jnp.dot(jnp.concatenate([pts, feat], axis=1), w1,
                preferred_element_type=jnp.float32)
    Z = jnp.dot(nx, w1[0:3, :],
                preferred_element_type=jnp.float32)

    s_new = jnp.sum(nx * nx, axis=1, keepdims=True)
    s_src = jnp.sum(ptsT * ptsT, axis=0, keepdims=True)
    cross = jnp.dot(nx, ptsT, preferred_element_type=jnp.float32)
    sqd = (s_new + s_src) - 2.0 * cross

    mask = sqd <= r2
    r = mask.astype(jnp.int16)
    sh = 1
    while sh < N:
        r = r + jnp.concatenate(
            [jnp.zeros((S, sh), jnp.int16), r[:, : N - sh]], axis=1)
        sh *= 2
    cnt = r[:, N - 1: N].astype(jnp.int32)

    t = jnp.where(mask, r, jnp.int16(0))
    tb = t.astype(jnp.bfloat16)
    Yb = Y.astype(jnp.bfloat16)
    one_b = jnp.bfloat16(1.0)
    zero_b = jnp.bfloat16(0.0)
    GRP = 8
    Zg = jnp.concatenate([Z - b1] * GRP, axis=0)
    hs = []
    for k0 in range(0, K, GRP):
        oh = jnp.concatenate(
            [jnp.where(tb == jnp.bfloat16(k + 1), one_b, zero_b)
             for k in range(k0, k0 + GRP)], axis=0)
        g = jnp.dot(oh, Yb, preferred_element_type=jnp.float32)
        hs.append(jnp.maximum(g - Zg, 0.0))
    H = jnp.concatenate(hs, axis=0)

    H = jnp.maximum(
        jnp.dot(H.astype(jnp.bfloat16), w2_ref[...].astype(jnp.bfloat16),
                preferred_element_type=jnp.float32)
        + b2_ref[...], 0.0)
    H = jnp.maximum(
        jnp.dot(H.astype(jnp.bfloat16), w3_ref[...].astype(jnp.bfloat16),
                preferred_element_type=jnp.float32)
        + b3_ref[...], 0.0)
    C3 = H.shape[1]
    H = H.reshape(K, S, C3)
    kio = jax.lax.broadcasted_iota(jnp.int32, (K, S, 1), 0)
    out_ref[0] = jnp.max(jnp.where(cnt[None, :, :] > kio, H, 0.0), axis=0)


def _sa(pts, feats, new_xyz, layers, radius, K):
    B, N, _ = pts.shape
    Cf = feats.shape[2]
    S = new_xyz.shape[1]
    (w1, b1), (w2, b2), (w3, b3) = layers
    C3 = w3.shape[1]
    ptsT = jnp.transpose(pts, (0, 2, 1))
    return pl.pallas_call(
        functools.partial(_sa_body, r2=radius ** 2, K=K),
        grid=(B,),
        in_specs=[
            pl.BlockSpec((1, N, 3), lambda b: (b, 0, 0)),
            pl.BlockSpec((1, 3, N), lambda b: (b, 0, 0)),
            pl.BlockSpec((1, N, Cf), lambda b: (b, 0, 0)),
            pl.BlockSpec((1, S, 3), lambda b: (b, 0, 0)),
            pl.BlockSpec(w1.shape, lambda b: (0, 0)),
            pl.BlockSpec(b1.shape, lambda b: (0, 0)),
            pl.BlockSpec(w2.shape, lambda b: (0, 0)),
            pl.BlockSpec(b2.shape, lambda b: (0, 0)),
            pl.BlockSpec(w3.shape, lambda b: (0, 0)),
            pl.BlockSpec(b3.shape, lambda b: (0, 0)),
        ],
        out_specs=pl.BlockSpec((1, S, C3), lambda b: (b, 0, 0)),
        out_shape=jax.ShapeDtypeStruct((B, S, C3), jnp.float32),
    )(pts, ptsT, feats, new_xyz, w1, b1, w2, b2, w3, b3)


def _sa3_body(xyz_ref, feat_ref, w1_ref, b1_ref, w2_ref, b2_ref, w3_ref,
              b3_ref, out_ref, *, B, M):
    x = jnp.concatenate([xyz_ref[...], feat_ref[...]], axis=1)
    h = jnp.maximum(
        jnp.dot(x, w1_ref[...], preferred_element_type=jnp.float32)
        + b1_ref[...], 0.0)
    h = jnp.maximum(
        jnp.dot(h, w2_ref[...], preferred_element_type=jnp.float32)
        + b2_ref[...], 0.0)
    h = jnp.maximum(
        jnp.dot(h, w3_ref[...], preferred_element_type=jnp.float32)
        + b3_ref[...], 0.0)
    out_ref[...] = jnp.max(h.reshape(B, M, h.shape[1]), axis=1)


def _sa3(l_xyz, l_points, layers):
    B, M, _ = l_xyz.shape
    (w1, b1), (w2, b2), (w3, b3) = layers
    C3 = w3.shape[1]
    return pl.pallas_call(
        functools.partial(_sa3_body, B=B, M=M),
        out_shape=jax.ShapeDtypeStruct((B, C3), jnp.float32),
    )(l_xyz.reshape(B * M, 3), l_points.reshape(B * M, -1),
      w1, b1, w2, b2, w3, b3)


def kernel(xyz, params):
    B = xyz.shape[0]
    l0_xyz = jnp.transpose(xyz, (0, 2, 1))
    sa1 = _fold(params['sa1'])
    sa2 = _fold(params['sa2'])
    sa3 = _fold(params['sa3'])

    l1_xyz = _fps(xyz, 512)
    l1_points = _sa(l0_xyz, l0_xyz, l1_xyz, sa1, 0.2, 32)
    l2_xyz = _fps(jnp.transpose(l1_xyz, (0, 2, 1)), 128)
    l2_points = _sa(l1_xyz, l1_points, l2_xyz, sa2, 0.4, 64)
    return _sa3(l2_xyz, l2_points, sa3)
